# baseline (speedup 1.0000x reference)
import dataclasses
import functools

import jax
import jax.numpy as jnp
from jax import lax
from jax.experimental import pallas as pl
from jax.experimental.pallas import tpu as pltpu
from jax.experimental.pallas import tpu_sc as plsc

N = 32
NN = N * N
L = 16
NCHUNK = NN // L
BIG = 3e38

f32 = jnp.float32
i32 = jnp.int32


def _lane():
    return lax.iota(i32, L)


def _clamp(idx):
    return jnp.clip(idx, 0, NN - 1)


def _gather(ref, idx):
    return plsc.load_gather(ref, [_clamp(idx)])


def _cummin(x):
    return -plsc.cummax(-x)


def _sc_body(img_hbm, e_hbm, uf_hbm, hf_hbm,
             img_v, pos_v, te_v, tse_v, ts_v, tsw_v,
             we_v, wse_v, ws_v, wsw_v,
             t_v, et_v, etn_v, u_v, mu_v, h_v, e_v):
    cid = lax.axis_index("c")
    sid = lax.axis_index("s")
    is_a = jnp.logical_and(cid == 0, sid == 0)
    is_b = jnp.logical_and(cid == 0, sid == 1)
    lane = _lane()

    @pl.when(jnp.logical_or(is_a, is_b))
    def _precompute():
        pltpu.sync_copy(img_hbm, img_v)

        @pl.loop(0, NCHUNK)
        def _sig(c):
            base = c * L
            x = img_v[pl.ds(base, L)]
            pos_v[pl.ds(base, L)] = 1.0 / (1.0 + jnp.exp(-x))

        @pl.loop(0, NCHUNK)
        def _theta(c):
            base = c * L
            idx = base + lane
            j = (c & 1) * L + lane
            ivec = jnp.full((L,), c >> 1, i32)
            m_e = j < N - 1
            m_s = ivec < N - 1
            p = pos_v[pl.ds(base, L)]
            zero = jnp.zeros((L,), f32)
            pe = _gather(pos_v, idx + 1)
            pse = _gather(pos_v, idx + N + 1)
            ps = _gather(pos_v, idx + N)
            psw = _gather(pos_v, idx + N - 1)
            de, dse, ds_, dsw = p - pe, p - pse, p - ps, p - psw
            te = jnp.where(m_e, de * de, zero)
            tse = jnp.where(jnp.logical_and(m_e, m_s), dse * dse, zero)
            ts = jnp.where(m_s, ds_ * ds_, zero)
            tsw = jnp.where(jnp.logical_and(j > 0, m_s), dsw * dsw, zero)
            sl = pl.ds(base, L)
            te_v[sl] = te
            tse_v[sl] = tse
            ts_v[sl] = ts
            tsw_v[sl] = tsw
            we_v[sl] = jnp.exp(-te)
            wse_v[sl] = jnp.exp(-tse)
            ws_v[sl] = jnp.exp(-ts)
            wsw_v[sl] = jnp.exp(-tsw)

        @pl.loop(0, N)
        def _prefix(i):
            base = i * N
            idx0 = base + lane
            s0 = jnp.where(lane > 0, _gather(te_v, idx0 - 1), 0.0)
            s1 = te_v[pl.ds(base + L - 1, L)]
            t0 = plsc.cumsum(s0)
            t1 = plsc.cumsum(s1) + jnp.sum(s0)
            t_v[pl.ds(base, L)] = t0
            t_v[pl.ds(base + L, L)] = t1
            et_v[pl.ds(base, L)] = jnp.exp(t0)
            et_v[pl.ds(base + L, L)] = jnp.exp(t1)
            etn_v[pl.ds(base, L)] = jnp.exp(-t0)
            etn_v[pl.ds(base + L, L)] = jnp.exp(-t1)

    @pl.when(is_a)
    def _soft():
        u_v[pl.ds(0, L)] = etn_v[pl.ds(0, L)]
        u_v[pl.ds(L, L)] = etn_v[pl.ds(L, L)]

        @pl.loop(1, N)
        def _fwd(i):
            base = i * N
            qs = []
            for c in range(2):
                idx = base + c * L + lane
                j = c * L + lane
                q = _gather(u_v, idx - N) * _gather(ws_v, idx - N)
                q = q + jnp.where(j > 0,
                                  _gather(u_v, idx - N - 1)
                                  * _gather(wse_v, idx - N - 1), 0.0)
                q = q + jnp.where(j < N - 1,
                                  _gather(u_v, idx - N + 1)
                                  * _gather(wsw_v, idx - N + 1), 0.0)
                qs.append(q * et_v[pl.ds(base + c * L, L)])
            g0, g1 = qs
            m = jnp.maximum(jnp.max(g0), jnp.max(g1))
            d0 = g0 / m
            d1 = g1 / m
            s0 = plsc.cumsum(d0)
            s1 = plsc.cumsum(d1) + jnp.sum(d0)
            u_v[pl.ds(base, L)] = etn_v[pl.ds(base, L)] * s0 * m
            u_v[pl.ds(base + L, L)] = etn_v[pl.ds(base + L, L)] * s1 * m

        pltpu.sync_copy(u_v.at[pl.ds(NN - L, L)], uf_hbm)

        @pl.loop(0, N)
        def _bwd(t):
            i = (N - 1) - t
            base = i * N
            rs = []
            for c in range(2):
                idx = base + c * L + lane
                j = c * L + lane
                sl = pl.ds(base + c * L, L)
                u_c = u_v[sl]
                qb = jnp.where(
                    j < N - 1,
                    _gather(mu_v, idx + N + 1) * wse_v[sl]
                    / _gather(u_v, idx + N + 1), 0.0)
                qb = qb + (_gather(mu_v, idx + N) * ws_v[sl]
                           / _gather(u_v, idx + N))
                qb = qb + jnp.where(
                    j > 0,
                    _gather(mu_v, idx + N - 1) * wsw_v[sl]
                    / _gather(u_v, idx + N - 1), 0.0)
                qb = u_c * qb
                qb = jnp.where(i == N - 1,
                               jnp.where(j == N - 1, 1.0, 0.0), qb)
                rs.append(qb * etn_v[sl] / u_c)
            r0, r1 = rs
            m = jnp.maximum(jnp.max(r0), jnp.max(r1))
            d0 = r0 / m
            d1 = r1 / m
            rc1 = lax.rev(plsc.cumsum(lax.rev(d1, (0,))), (0,))
            rc0 = lax.rev(plsc.cumsum(lax.rev(d0, (0,))), (0,)) + jnp.sum(d1)
            sl0 = pl.ds(base, L)
            sl1 = pl.ds(base + L, L)
            mu_v[sl0] = u_v[sl0] * et_v[sl0] * rc0 * m
            mu_v[sl1] = u_v[sl1] * et_v[sl1] * rc1 * m

        @pl.loop(0, NCHUNK)
        def _eout(c):
            base = c * L
            idx = base + lane
            j = (c & 1) * L + lane
            ivec = jnp.full((L,), c >> 1, i32)
            sl = pl.ds(base, L)
            u_c = u_v[sl]
            m_e = j < N - 1
            m_s = ivec < N - 1
            e0 = jnp.where(m_e, _gather(mu_v, idx + 1) * u_c * we_v[sl]
                           / _gather(u_v, idx + 1), 0.0)
            e1 = jnp.where(jnp.logical_and(m_e, m_s),
                           _gather(mu_v, idx + N + 1) * u_c * wse_v[sl]
                           / _gather(u_v, idx + N + 1), 0.0)
            e2 = jnp.where(m_s, _gather(mu_v, idx + N) * u_c * ws_v[sl]
                           / _gather(u_v, idx + N), 0.0)
            e3 = jnp.where(jnp.logical_and(j > 0, m_s),
                           _gather(mu_v, idx + N - 1) * u_c * wsw_v[sl]
                           / _gather(u_v, idx + N - 1), 0.0)
            for d, e in enumerate((e0, e1, e2, e3)):
                plsc.store_scatter(e_v, [idx * 4 + d], e)

        pltpu.sync_copy(e_v, e_hbm)

    @pl.when(is_b)
    def _hard():
        h_v[pl.ds(0, L)] = t_v[pl.ds(0, L)]
        h_v[pl.ds(L, L)] = t_v[pl.ds(L, L)]

        @pl.loop(1, N)
        def _hdp(i):
            base = i * N
            bs = []
            for c in range(2):
                idx = base + c * L + lane
                j = c * L + lane
                b = _gather(h_v, idx - N) + _gather(ts_v, idx - N)
                b = jnp.minimum(b, jnp.where(
                    j > 0,
                    _gather(h_v, idx - N - 1) + _gather(tse_v, idx - N - 1),
                    BIG))
                b = jnp.minimum(b, jnp.where(
                    j < N - 1,
                    _gather(h_v, idx - N + 1) + _gather(tsw_v, idx - N + 1),
                    BIG))
                bs.append(b - t_v[pl.ds(base + c * L, L)])
            x0, x1 = bs
            cm0 = _cummin(x0)
            cm1 = jnp.minimum(_cummin(x1), jnp.min(x0))
            h_v[pl.ds(base, L)] = t_v[pl.ds(base, L)] + cm0
            h_v[pl.ds(base + L, L)] = t_v[pl.ds(base + L, L)] + cm1

        pltpu.sync_copy(h_v.at[pl.ds(NN - L, L)], hf_hbm)


def _compiler_params():
    cp = pltpu.CompilerParams()
    if "needs_layout_passes" in pltpu.CompilerParams.__dataclass_fields__:
        cp = dataclasses.replace(cp, needs_layout_passes=False)
    return cp


@functools.partial(jax.jit, static_argnames=())
def kernel(image):
    mesh = plsc.VectorSubcoreMesh(core_axis_name="c", subcore_axis_name="s",
                                  num_cores=2, num_subcores=16)
    vec = lambda n: pltpu.VMEM((n,), f32)
    run = pl.kernel(
        _sc_body,
        out_type=(
            jax.ShapeDtypeStruct((4 * NN,), f32),
            jax.ShapeDtypeStruct((L,), f32),
            jax.ShapeDtypeStruct((L,), f32),
        ),
        mesh=mesh,
        scratch_types=[vec(NN) for _ in range(16)] + [vec(4 * NN)],
        compiler_params=_compiler_params(),
    )
    e_flat, u_tail, h_tail = run(image.reshape(-1).astype(f32))
    v = -jnp.log(u_tail[L - 1])
    v_hard = h_tail[L - 1]
    return (v, e_flat.reshape(NN, 4), v_hard)


# speedup vs baseline: 1.0479x; 1.0479x over previous
import dataclasses
import functools

import jax
import jax.numpy as jnp
from jax import lax
from jax.experimental import pallas as pl
from jax.experimental.pallas import tpu as pltpu
from jax.experimental.pallas import tpu_sc as plsc

N = 32
NN = N * N
L = 16
NCHUNK = NN // L
BIG = 3e38

f32 = jnp.float32
i32 = jnp.int32


def _lane():
    return lax.iota(i32, L)


def _clamp(idx):
    return jnp.clip(idx, 0, NN - 1)


def _gather(ref, idx):
    return plsc.load_gather(ref, [_clamp(idx)])


def _cummin(x):
    return -plsc.cummax(-x)


def _sc_body(img_hbm, e_hbm, uf_hbm, hf_hbm,
             img_v, pos_v, te_v, tse_v, ts_v, tsw_v,
             we_v, wse_v, ws_v, wsw_v,
             t_v, et_v, etn_v, u_v, mu_v, h_v, e_v):
    cid = lax.axis_index("c")
    sid = lax.axis_index("s")
    is_a = jnp.logical_and(cid == 0, sid == 0)
    is_b = jnp.logical_and(cid == 0, sid == 1)
    lane = _lane()

    @pl.when(jnp.logical_or(is_a, is_b))
    def _precompute():
        pltpu.sync_copy(img_hbm, img_v)

        @pl.loop(0, NCHUNK)
        def _sig(c):
            base = c * L
            x = img_v[pl.ds(base, L)]
            pos_v[pl.ds(base, L)] = 1.0 / (1.0 + jnp.exp(-x))

        @pl.loop(0, NCHUNK)
        def _theta(c):
            base = c * L
            idx = base + lane
            j = (c & 1) * L + lane
            ivec = jnp.full((L,), c >> 1, i32)
            m_e = j < N - 1
            m_s = ivec < N - 1
            p = pos_v[pl.ds(base, L)]
            zero = jnp.zeros((L,), f32)
            pe = _gather(pos_v, idx + 1)
            pse = _gather(pos_v, idx + N + 1)
            ps = _gather(pos_v, idx + N)
            psw = _gather(pos_v, idx + N - 1)
            de, dse, ds_, dsw = p - pe, p - pse, p - ps, p - psw
            te = jnp.where(m_e, de * de, zero)
            tse = jnp.where(jnp.logical_and(m_e, m_s), dse * dse, zero)
            ts = jnp.where(m_s, ds_ * ds_, zero)
            tsw = jnp.where(jnp.logical_and(j > 0, m_s), dsw * dsw, zero)
            sl = pl.ds(base, L)
            te_v[sl] = te
            tse_v[sl] = tse
            ts_v[sl] = ts
            tsw_v[sl] = tsw
            we_v[sl] = jnp.exp(-te)
            wse_v[sl] = jnp.exp(-tse)
            ws_v[sl] = jnp.exp(-ts)
            wsw_v[sl] = jnp.exp(-tsw)

        @pl.loop(0, N)
        def _prefix(i):
            base = i * N
            idx0 = base + lane
            s0 = jnp.where(lane > 0, _gather(te_v, idx0 - 1), 0.0)
            s1 = te_v[pl.ds(base + L - 1, L)]
            t0 = plsc.cumsum(s0)
            t1 = plsc.cumsum(s1) + jnp.sum(s0)
            t_v[pl.ds(base, L)] = t0
            t_v[pl.ds(base + L, L)] = t1
            et_v[pl.ds(base, L)] = jnp.exp(t0)
            et_v[pl.ds(base + L, L)] = jnp.exp(t1)
            etn_v[pl.ds(base, L)] = jnp.exp(-t0)
            etn_v[pl.ds(base + L, L)] = jnp.exp(-t1)

    @pl.when(is_a)
    def _soft():
        u_v[pl.ds(0, L)] = etn_v[pl.ds(0, L)]
        u_v[pl.ds(L, L)] = etn_v[pl.ds(L, L)]

        @pl.loop(1, N)
        def _fwd(i):
            base = i * N
            qs = []
            for c in range(2):
                idx = base + c * L + lane
                j = c * L + lane
                q = _gather(u_v, idx - N) * _gather(ws_v, idx - N)
                q = q + jnp.where(j > 0,
                                  _gather(u_v, idx - N - 1)
                                  * _gather(wse_v, idx - N - 1), 0.0)
                q = q + jnp.where(j < N - 1,
                                  _gather(u_v, idx - N + 1)
                                  * _gather(wsw_v, idx - N + 1), 0.0)
                qs.append(q * et_v[pl.ds(base + c * L, L)])
            g0, g1 = qs
            m = jnp.maximum(jnp.max(g0), jnp.max(g1))
            d0 = g0 / m
            d1 = g1 / m
            s0 = plsc.cumsum(d0)
            s1 = plsc.cumsum(d1) + jnp.sum(d0)
            u_v[pl.ds(base, L)] = etn_v[pl.ds(base, L)] * s0 * m
            u_v[pl.ds(base + L, L)] = etn_v[pl.ds(base + L, L)] * s1 * m

        pltpu.sync_copy(u_v.at[pl.ds(NN - L, L)], uf_hbm)

        @pl.loop(0, N)
        def _bwd(t):
            i = (N - 1) - t
            base = i * N
            rs = []
            for c in range(2):
                idx = base + c * L + lane
                j = c * L + lane
                sl = pl.ds(base + c * L, L)
                u_c = u_v[sl]
                qb = jnp.where(
                    j < N - 1,
                    _gather(mu_v, idx + N + 1) * wse_v[sl]
                    / _gather(u_v, idx + N + 1), 0.0)
                qb = qb + (_gather(mu_v, idx + N) * ws_v[sl]
                           / _gather(u_v, idx + N))
                qb = qb + jnp.where(
                    j > 0,
                    _gather(mu_v, idx + N - 1) * wsw_v[sl]
                    / _gather(u_v, idx + N - 1), 0.0)
                qb = u_c * qb
                qb = jnp.where(i == N - 1,
                               jnp.where(j == N - 1, 1.0, 0.0), qb)
                rs.append(qb * etn_v[sl] / u_c)
            r0, r1 = rs
            m = jnp.maximum(jnp.max(r0), jnp.max(r1))
            d0 = r0 / m
            d1 = r1 / m
            rc1 = lax.rev(plsc.cumsum(lax.rev(d1, (0,))), (0,))
            rc0 = lax.rev(plsc.cumsum(lax.rev(d0, (0,))), (0,)) + jnp.sum(d1)
            sl0 = pl.ds(base, L)
            sl1 = pl.ds(base + L, L)
            mu_v[sl0] = u_v[sl0] * et_v[sl0] * rc0 * m
            mu_v[sl1] = u_v[sl1] * et_v[sl1] * rc1 * m

        @pl.loop(0, NCHUNK)
        def _eout(c):
            base = c * L
            idx = base + lane
            j = (c & 1) * L + lane
            ivec = jnp.full((L,), c >> 1, i32)
            sl = pl.ds(base, L)
            u_c = u_v[sl]
            m_e = j < N - 1
            m_s = ivec < N - 1
            e0 = jnp.where(m_e, _gather(mu_v, idx + 1) * u_c * we_v[sl]
                           / _gather(u_v, idx + 1), 0.0)
            e1 = jnp.where(jnp.logical_and(m_e, m_s),
                           _gather(mu_v, idx + N + 1) * u_c * wse_v[sl]
                           / _gather(u_v, idx + N + 1), 0.0)
            e2 = jnp.where(m_s, _gather(mu_v, idx + N) * u_c * ws_v[sl]
                           / _gather(u_v, idx + N), 0.0)
            e3 = jnp.where(jnp.logical_and(j > 0, m_s),
                           _gather(mu_v, idx + N - 1) * u_c * wsw_v[sl]
                           / _gather(u_v, idx + N - 1), 0.0)
            for d, e in enumerate((e0, e1, e2, e3)):
                plsc.store_scatter(e_v, [idx * 4 + d], e)

        pltpu.sync_copy(e_v, e_hbm)

    @pl.when(is_b)
    def _hard():
        h_v[pl.ds(0, L)] = t_v[pl.ds(0, L)]
        h_v[pl.ds(L, L)] = t_v[pl.ds(L, L)]

        @pl.loop(1, N)
        def _hdp(i):
            base = i * N
            bs = []
            for c in range(2):
                idx = base + c * L + lane
                j = c * L + lane
                b = _gather(h_v, idx - N) + _gather(ts_v, idx - N)
                b = jnp.minimum(b, jnp.where(
                    j > 0,
                    _gather(h_v, idx - N - 1) + _gather(tse_v, idx - N - 1),
                    BIG))
                b = jnp.minimum(b, jnp.where(
                    j < N - 1,
                    _gather(h_v, idx - N + 1) + _gather(tsw_v, idx - N + 1),
                    BIG))
                bs.append(b - t_v[pl.ds(base + c * L, L)])
            x0, x1 = bs
            cm0 = _cummin(x0)
            cm1 = jnp.minimum(_cummin(x1), jnp.min(x0))
            h_v[pl.ds(base, L)] = t_v[pl.ds(base, L)] + cm0
            h_v[pl.ds(base + L, L)] = t_v[pl.ds(base + L, L)] + cm1

        pltpu.sync_copy(h_v.at[pl.ds(NN - L, L)], hf_hbm)


def _compiler_params():
    cp = pltpu.CompilerParams()
    if "needs_layout_passes" in pltpu.CompilerParams.__dataclass_fields__:
        cp = dataclasses.replace(cp, needs_layout_passes=False)
    return cp


@functools.partial(jax.jit, static_argnames=())
def kernel(image):
    mesh = plsc.VectorSubcoreMesh(core_axis_name="c", subcore_axis_name="s",
                                  num_cores=1, num_subcores=16)
    vec = lambda n: pltpu.VMEM((n,), f32)
    run = pl.kernel(
        _sc_body,
        out_type=(
            jax.ShapeDtypeStruct((4 * NN,), f32),
            jax.ShapeDtypeStruct((L,), f32),
            jax.ShapeDtypeStruct((L,), f32),
        ),
        mesh=mesh,
        scratch_types=[vec(NN) for _ in range(16)] + [vec(4 * NN)],
        compiler_params=_compiler_params(),
    )
    e_flat, u_tail, h_tail = run(image.reshape(-1).astype(f32))
    v = -jnp.log(u_tail[L - 1])
    v_hard = h_tail[L - 1]
    return (v, e_flat.reshape(NN, 4), v_hard)


# speedup vs baseline: 1.0622x; 1.0137x over previous
import dataclasses
import functools

import jax
import jax.numpy as jnp
from jax import lax
from jax.experimental import pallas as pl
from jax.experimental.pallas import tpu as pltpu
from jax.experimental.pallas import tpu_sc as plsc

N = 32
NN = N * N
L = 16
NCHUNK = NN // L
BIG = 3e38

f32 = jnp.float32
i32 = jnp.int32


def _lane():
    return lax.iota(i32, L)


def _clamp(idx):
    return jnp.clip(idx, 0, NN - 1)


def _gather(ref, idx):
    return plsc.load_gather(ref, [_clamp(idx)])


def _cummin(x):
    return -plsc.cummax(-x)


def _revcum(x):
    return lax.rev(plsc.cumsum(lax.rev(x, (0,))), (0,))


def _sc_body(img_hbm, e_hbm, uf_hbm, hf_hbm,
             img_v, pos_v, te_v, tse_v, ts_v, tsw_v,
             we_v, wse_v, ws_v, wsw_v,
             t_v, et_v, etn_v, u_v, f_v, h_v, e_v):
    cid = lax.axis_index("c")
    sid = lax.axis_index("s")
    is_a = jnp.logical_and(cid == 0, sid == 0)
    is_b = jnp.logical_and(cid == 0, sid == 1)
    lane = _lane()

    def _sigmoid():
        pltpu.sync_copy(img_hbm, img_v)

        @pl.loop(0, NCHUNK)
        def _sig(c):
            base = c * L
            x = img_v[pl.ds(base, L)]
            pos_v[pl.ds(base, L)] = 1.0 / (1.0 + jnp.exp(-x))

    def _theta_chunk(c):
        base = c * L
        idx = base + lane
        j = (c & 1) * L + lane
        ivec = jnp.full((L,), c >> 1, i32)
        m_e = j < N - 1
        m_s = ivec < N - 1
        p = pos_v[pl.ds(base, L)]
        zero = jnp.zeros((L,), f32)
        pe = _gather(pos_v, idx + 1)
        pse = _gather(pos_v, idx + N + 1)
        ps = _gather(pos_v, idx + N)
        psw = _gather(pos_v, idx + N - 1)
        de, dse, ds_, dsw = p - pe, p - pse, p - ps, p - psw
        te = jnp.where(m_e, de * de, zero)
        tse = jnp.where(jnp.logical_and(m_e, m_s), dse * dse, zero)
        ts = jnp.where(m_s, ds_ * ds_, zero)
        tsw = jnp.where(jnp.logical_and(j > 0, m_s), dsw * dsw, zero)
        return te, tse, ts, tsw

    def _prefix_shift(i):
        base = i * N
        idx0 = base + lane
        s0 = jnp.where(lane > 0, _gather(te_v, idx0 - 1), 0.0)
        s1 = te_v[pl.ds(base + L - 1, L)]
        t0 = plsc.cumsum(s0)
        t1 = plsc.cumsum(s1) + jnp.sum(s0)
        return t0, t1

    @pl.when(is_a)
    def _tile_a():
        _sigmoid()

        @pl.loop(0, NCHUNK)
        def _theta(c):
            te, tse, ts, tsw = _theta_chunk(c)
            sl = pl.ds(c * L, L)
            te_v[sl] = te
            we_v[sl] = jnp.exp(-te)
            wse_v[sl] = jnp.exp(-tse)
            ws_v[sl] = jnp.exp(-ts)
            wsw_v[sl] = jnp.exp(-tsw)

        @pl.loop(0, N)
        def _prefix(i):
            base = i * N
            t0, t1 = _prefix_shift(i)
            et_v[pl.ds(base, L)] = jnp.exp(t0)
            et_v[pl.ds(base + L, L)] = jnp.exp(t1)
            etn_v[pl.ds(base, L)] = jnp.exp(-t0)
            etn_v[pl.ds(base + L, L)] = jnp.exp(-t1)

        u_v[pl.ds(0, L)] = etn_v[pl.ds(0, L)]
        u_v[pl.ds(L, L)] = etn_v[pl.ds(L, L)]

        @pl.loop(1, N)
        def _fwd(i):
            base = i * N
            qs = []
            for c in range(2):
                idx = base + c * L + lane
                j = c * L + lane
                q = _gather(u_v, idx - N) * _gather(ws_v, idx - N)
                q = q + jnp.where(j > 0,
                                  _gather(u_v, idx - N - 1)
                                  * _gather(wse_v, idx - N - 1), 0.0)
                q = q + jnp.where(j < N - 1,
                                  _gather(u_v, idx - N + 1)
                                  * _gather(wsw_v, idx - N + 1), 0.0)
                qs.append(q * et_v[pl.ds(base + c * L, L)])
            g0, g1 = qs
            m = jnp.maximum(jnp.max(g0), jnp.max(g1))
            d0 = g0 / m
            d1 = g1 / m
            s0 = plsc.cumsum(d0)
            s1 = plsc.cumsum(d1) + jnp.sum(d0)
            u_v[pl.ds(base, L)] = etn_v[pl.ds(base, L)] * s0 * m
            u_v[pl.ds(base + L, L)] = etn_v[pl.ds(base + L, L)] * s1 * m

        pltpu.sync_copy(u_v.at[pl.ds(NN - L, L)], uf_hbm)

        def _store_f_and_east(base, rs0, rs1):
            m = jnp.maximum(jnp.max(rs0), jnp.max(rs1))
            d0 = rs0 / m
            d1 = rs1 / m
            rc1 = _revcum(d1)
            rc0 = _revcum(d0) + jnp.sum(d1)
            sl0 = pl.ds(base, L)
            sl1 = pl.ds(base + L, L)
            f_v[sl0] = et_v[sl0] * rc0 * m
            f_v[sl1] = et_v[sl1] * rc1 * m
            for c in range(2):
                idx = base + c * L + lane
                j = c * L + lane
                sl = pl.ds(base + c * L, L)
                e0 = jnp.where(j < N - 1,
                               u_v[sl] * we_v[sl] * _gather(f_v, idx + 1),
                               0.0)
                plsc.store_scatter(e_v, [idx * 4], e0)

        base_l = NN - N
        sl1_l = pl.ds(base_l + L, L)
        rs1_l = jnp.where(lane == L - 1, etn_v[sl1_l] / u_v[sl1_l], 0.0)
        _store_f_and_east(base_l, jnp.zeros((L,), f32), rs1_l)
        zero_l = jnp.zeros((L,), f32)
        for c in range(2):
            idx = base_l + c * L + lane
            for d in (1, 2, 3):
                plsc.store_scatter(e_v, [idx * 4 + d], zero_l)

        @pl.loop(1, N)
        def _bwd(t):
            i = (N - 1) - t
            base = i * N
            rs = []
            for c in range(2):
                idx = base + c * L + lane
                j = c * L + lane
                sl = pl.ds(base + c * L, L)
                u_c = u_v[sl]
                t1 = jnp.where(j < N - 1,
                               _gather(f_v, idx + N + 1) * wse_v[sl], 0.0)
                t2 = _gather(f_v, idx + N) * ws_v[sl]
                t3 = jnp.where(j > 0,
                               _gather(f_v, idx + N - 1) * wsw_v[sl], 0.0)
                plsc.store_scatter(e_v, [idx * 4 + 1], u_c * t1)
                plsc.store_scatter(e_v, [idx * 4 + 2], u_c * t2)
                plsc.store_scatter(e_v, [idx * 4 + 3], u_c * t3)
                rs.append((t1 + t2 + t3) * etn_v[sl])
            _store_f_and_east(base, rs[0], rs[1])

        pltpu.sync_copy(e_v, e_hbm)

    @pl.when(is_b)
    def _tile_b():
        _sigmoid()

        @pl.loop(0, NCHUNK)
        def _theta(c):
            te, tse, ts, tsw = _theta_chunk(c)
            sl = pl.ds(c * L, L)
            te_v[sl] = te
            tse_v[sl] = tse
            ts_v[sl] = ts
            tsw_v[sl] = tsw

        @pl.loop(0, N)
        def _prefix(i):
            base = i * N
            t0, t1 = _prefix_shift(i)
            t_v[pl.ds(base, L)] = t0
            t_v[pl.ds(base + L, L)] = t1

        h_v[pl.ds(0, L)] = t_v[pl.ds(0, L)]
        h_v[pl.ds(L, L)] = t_v[pl.ds(L, L)]

        @pl.loop(1, N)
        def _hdp(i):
            base = i * N
            bs = []
            for c in range(2):
                idx = base + c * L + lane
                j = c * L + lane
                b = _gather(h_v, idx - N) + _gather(ts_v, idx - N)
                b = jnp.minimum(b, jnp.where(
                    j > 0,
                    _gather(h_v, idx - N - 1) + _gather(tse_v, idx - N - 1),
                    BIG))
                b = jnp.minimum(b, jnp.where(
                    j < N - 1,
                    _gather(h_v, idx - N + 1) + _gather(tsw_v, idx - N + 1),
                    BIG))
                bs.append(b - t_v[pl.ds(base + c * L, L)])
            x0, x1 = bs
            cm0 = _cummin(x0)
            cm1 = jnp.minimum(_cummin(x1), jnp.min(x0))
            h_v[pl.ds(base, L)] = t_v[pl.ds(base, L)] + cm0
            h_v[pl.ds(base + L, L)] = t_v[pl.ds(base + L, L)] + cm1

        pltpu.sync_copy(h_v.at[pl.ds(NN - L, L)], hf_hbm)


def _compiler_params():
    cp = pltpu.CompilerParams()
    if "needs_layout_passes" in pltpu.CompilerParams.__dataclass_fields__:
        cp = dataclasses.replace(cp, needs_layout_passes=False)
    return cp


@functools.partial(jax.jit, static_argnames=())
def kernel(image):
    mesh = plsc.VectorSubcoreMesh(core_axis_name="c", subcore_axis_name="s",
                                  num_cores=1, num_subcores=16)
    vec = lambda n: pltpu.VMEM((n,), f32)
    run = pl.kernel(
        _sc_body,
        out_type=(
            jax.ShapeDtypeStruct((4 * NN,), f32),
            jax.ShapeDtypeStruct((L,), f32),
            jax.ShapeDtypeStruct((L,), f32),
        ),
        mesh=mesh,
        scratch_types=[vec(NN) for _ in range(16)] + [vec(4 * NN)],
        compiler_params=_compiler_params(),
    )
    e_flat, u_tail, h_tail = run(image.reshape(-1).astype(f32))
    v = -jnp.log(u_tail[L - 1])
    v_hard = h_tail[L - 1]
    return (v, e_flat.reshape(NN, 4), v_hard)
